# Initial kernel scaffold; baseline (speedup 1.0000x reference)
import jax
import jax.numpy as jnp
from jax.experimental import pallas as pl


def kernel(graph_feat, edge, weight, position, att_W, att_b, norm_g, norm_b, fuse_W, fuse_b, fn_g, fn_b, d1_W, d1_b, d2_W, d2_b, dn_g, dn_b):
    raise NotImplementedError("write your pallas kernel here")


# speedup vs baseline: 2.1389x; 2.1389x over previous
import functools

import jax
import jax.numpy as jnp
from jax import lax
from jax.experimental import pallas as pl
from jax.experimental.pallas import tpu as pltpu
from jax.experimental.pallas import tpu_sc as plsc

NC = 2
NS = 16
NW = NC * NS
CH = 128
PADROWS = 512
DUPROWS = 128


def _gn(h, g, b, eps=1e-5):
    mu = jnp.mean(h, axis=-1, keepdims=True)
    d = h - mu
    var = jnp.mean(d * d, axis=-1, keepdims=True)
    return d * lax.rsqrt(var + eps) * g + b


def _lrelu(h):
    return jnp.where(h >= 0, h, 0.01 * h)


def _tc_edge_body(v_ref, pdx_ref, pdy_ref, w_ref,
                  d1t_ref, d1b_ref, d2t_ref, d2b_ref, dng_ref, dnb_ref,
                  u_ref, ax_ref):
    be = v_ref.shape[0]
    pdx = pdx_ref[...]
    pdy = pdy_ref[...]
    a = pdx * d1t_ref[0:1, :] + pdy * d1t_ref[1:2, :] + d1b_ref[...]
    h1 = _lrelu(a)
    h2 = jnp.dot(h1, d2t_ref[...], preferred_element_type=jnp.float32)
    h2 = h2 + d2b_ref[...]
    de = _gn(h2, dng_ref[...], dnb_ref[...])
    w = w_ref[...]
    u = w * (v_ref[...] + de)
    lane = lax.broadcasted_iota(jnp.int32, (be, 32), 1)
    aux = jnp.where(lane == 0, w, jnp.where(lane == 1, 1.0, 0.0))
    u_ref[...] = u
    ax_ref[...] = aux


def _tc_node_body(x_ref, s_ref, aux_ref,
                  attt_ref, attb_ref, ng_ref, nb_ref, ftx_ref, ftc_ref,
                  fb_ref, fng_ref, fnb_ref, out_ref):
    x = x_ref[...]
    aux = aux_ref[...]
    ws = aux[:, 0:1]
    cnt = aux[:, 1:2]
    s = s_ref[...]
    copy = jnp.dot(s, attt_ref[...], preferred_element_type=jnp.float32)
    copy = copy + ws * attb_ref[...]
    copy = _gn(copy, ng_ref[...], nb_ref[...])
    f = (jnp.dot(x, ftx_ref[...], preferred_element_type=jnp.float32)
         + jnp.dot(copy, ftc_ref[...], preferred_element_type=jnp.float32)
         + fb_ref[...])
    f = _lrelu(_gn(f, fng_ref[...], fnb_ref[...]))
    out_ref[...] = jnp.where(cnt > 0, f, x)


def _sc_gather_body(epad, x_hbm, ptab_hbm, pre_hbm, ip_hbm, is_hbm,
                    v_out, pdx_out, pdy_out,
                    ptab_v, preb, ipb, isb, vbuf, pdxb, pdyb, sem0):
    ew = epad // NW
    nchunk = ew // CH
    c = lax.axis_index("c")
    s = lax.axis_index("s")
    base = (s * NC + c) * ew

    pltpu.sync_copy(ptab_hbm, ptab_v)

    def body(k, carry):
        off = base + k * CH
        pltpu.sync_copy(pre_hbm.at[pl.ds(off, CH)], preb)
        pltpu.sync_copy(ip_hbm.at[pl.ds(off, CH)], ipb)
        pltpu.sync_copy(is_hbm.at[pl.ds(off, CH)], isb)
        cp0 = pltpu.async_copy(x_hbm.at[preb], vbuf, sem0)
        for g in range(CH // 16):
            ip_v = ipb[pl.ds(g * 16, 16)] * 2
            is_v = isb[pl.ds(g * 16, 16)] * 2
            ppx = plsc.load_gather(ptab_v, [ip_v])
            ppy = plsc.load_gather(ptab_v, [ip_v + 1])
            psx = plsc.load_gather(ptab_v, [is_v])
            psy = plsc.load_gather(ptab_v, [is_v + 1])
            pdxb[pl.ds(g * 16, 16)] = psx - ppx
            pdyb[pl.ds(g * 16, 16)] = psy - ppy
        pltpu.sync_copy(pdxb, pdx_out.at[pl.ds(off, CH)])
        pltpu.sync_copy(pdyb, pdy_out.at[pl.ds(off, CH)])
        cp0.wait()
        pltpu.sync_copy(vbuf, v_out.at[pl.ds(off, CH)])
        return carry

    lax.fori_loop(0, nchunk, body, 0)


def _sc_scatter_body(epad, nacc, n, u0, u1, u2, u3, ax, sm_hbm, zr_hbm,
                     s0_out, s1_out, s2_out, s3_out, auxa_out, auxb_out,
                     acc, sidx, jidx, ubuf, jbuf):
    ew = epad // NS
    nchunk = ew // CH
    ewh = epad // (2 * NS)
    nchunk_h = ewh // CH
    strz = nacc // NS
    strf = n // NS
    c = lax.axis_index("c")
    s = lax.axis_index("s")

    pltpu.sync_copy(sm_hbm.at[pl.ds(0, 8)], jidx.at[0])

    def zero_acc():
        pltpu.sync_copy(zr_hbm.at[pl.ds(s * strz, strz)],
                        acc.at[pl.ds(s * strz // 4, strz)])

    def chunk_loop(u_ref, base, nck):
        def body(k, carry):
            off = base + k * CH
            pltpu.sync_copy(sm_hbm.at[pl.ds(off, CH)], sidx.at[0])
            pltpu.sync_copy(u_ref.at[pl.ds(off, CH)], ubuf)
            pltpu.sync_copy(ubuf, acc.at[sidx.at[0]], add=True)
            pltpu.sync_copy(acc.at[jidx.at[0]], jbuf)
            return carry
        lax.fori_loop(0, nck, body, 0)

    def flush(dst):
        pltpu.sync_copy(acc.at[pl.ds(s * strf // 4, strf)],
                        dst.at[pl.ds(s * strf, strf)])

    def one_pass(u_ref0, u_ref1, out0, out1, base0, base1, nck):
        zero_acc()
        plsc.subcore_barrier()

        @pl.when(c == 0)
        def _():
            chunk_loop(u_ref0, base0, nck)

        @pl.when(c == 1)
        def _():
            chunk_loop(u_ref1, base1, nck)

        plsc.subcore_barrier()

        @pl.when(c == 0)
        def _():
            flush(out0)

        @pl.when(c == 1)
        def _():
            flush(out1)

        plsc.subcore_barrier()

    one_pass(u0, u2, s0_out, s2_out, s * ew, s * ew, nchunk)
    one_pass(u1, u3, s1_out, s3_out, s * ew, s * ew, nchunk)
    one_pass(ax, ax, auxa_out, auxb_out, s * ewh, epad // 2 + s * ewh,
             nchunk_h)


def kernel(graph_feat, edge, weight, position, att_W, att_b, norm_g, norm_b,
           fuse_W, fuse_b, fn_g, fn_b, d1_W, d1_b, d2_W, d2_b, dn_g, dn_b):
    B, T, HW, C = graph_feat.shape
    N = B * T * HW
    E = edge.shape[0]
    THW = T * HW
    f32 = jnp.float32

    x = graph_feat.reshape(N, C)

    eb = edge[:, 0]
    et = edge[:, 1]
    ep = edge[:, 2]
    es = edge[:, 3]
    pre = eb * THW + et * HW + ep
    suc = eb * THW + (et + 1) * HW + es
    ip = et * HW + ep
    isc = (et + 1) * HW + es

    epad = ((E + NW * CH - 1) // (NW * CH)) * (NW * CH)
    npad = epad - E
    dupbase = N + PADROWS
    nacc = ((N + PADROWS + DUPROWS + 63) // 64) * 64
    ar = jnp.arange(npad, dtype=jnp.int32)
    pre_p = jnp.concatenate([pre, (ar * 97) % N])
    suc_p = jnp.concatenate([suc, N + (ar % PADROWS)])
    ip_p = jnp.concatenate([ip, ar % THW])
    is_p = jnp.concatenate([isc, ar % THW])
    w_p = jnp.concatenate([weight, jnp.zeros((npad,), f32)])

    ptab_flat = position.reshape(THW * 2)

    mesh = plsc.VectorSubcoreMesh(core_axis_name="c", subcore_axis_name="s",
                                  num_cores=NC, num_subcores=NS)

    sc_gather = pl.kernel(
        functools.partial(_sc_gather_body, epad),
        out_type=[
            jax.ShapeDtypeStruct((epad, C), f32),
            jax.ShapeDtypeStruct((epad,), f32),
            jax.ShapeDtypeStruct((epad,), f32),
        ],
        mesh=mesh,
        compiler_params=pltpu.CompilerParams(needs_layout_passes=False),
        scratch_types=[
            pltpu.VMEM((THW * 2,), f32),
            pltpu.VMEM((CH,), jnp.int32),
            pltpu.VMEM((CH,), jnp.int32),
            pltpu.VMEM((CH,), jnp.int32),
            pltpu.VMEM((CH, C), f32),
            pltpu.VMEM((CH,), f32),
            pltpu.VMEM((CH,), f32),
            pltpu.SemaphoreType.DMA,
        ],
    )
    v_rows, pdx, pdy = sc_gather(x, ptab_flat, pre_p, ip_p, is_p)

    be = 1024
    grid_e = epad // be
    u_full, aux32 = pl.pallas_call(
        _tc_edge_body,
        grid=(grid_e,),
        in_specs=[
            pl.BlockSpec((be, C), lambda i: (i, 0)),
            pl.BlockSpec((be, 1), lambda i: (i, 0)),
            pl.BlockSpec((be, 1), lambda i: (i, 0)),
            pl.BlockSpec((be, 1), lambda i: (i, 0)),
            pl.BlockSpec((2, C), lambda i: (0, 0)),
            pl.BlockSpec((1, C), lambda i: (0, 0)),
            pl.BlockSpec((C, C), lambda i: (0, 0)),
            pl.BlockSpec((1, C), lambda i: (0, 0)),
            pl.BlockSpec((1, C), lambda i: (0, 0)),
            pl.BlockSpec((1, C), lambda i: (0, 0)),
        ],
        out_specs=[pl.BlockSpec((be, C), lambda i: (i, 0)),
                   pl.BlockSpec((be, 32), lambda i: (i, 0))],
        out_shape=[jax.ShapeDtypeStruct((epad, C), f32),
                   jax.ShapeDtypeStruct((epad, 32), f32)],
    )(v_rows, pdx.reshape(epad, 1), pdy.reshape(epad, 1),
      w_p.reshape(epad, 1), d1_W.T, d1_b.reshape(1, C), d2_W.T,
      d2_b.reshape(1, C), dn_g.reshape(1, C), dn_b.reshape(1, C))

    s_acc = jnp.zeros((nacc, C), f32).at[suc_p].add(u_full)[:N]
    ax_acc = jnp.zeros((nacc, 32), f32).at[suc_p].add(aux32)[:N]

    bn = 1024
    grid_n = N // bn
    fuse_t = fuse_W.T
    out = pl.pallas_call(
        _tc_node_body,
        grid=(grid_n,),
        in_specs=[
            pl.BlockSpec((bn, C), lambda i: (i, 0)),
            pl.BlockSpec((bn, C), lambda i: (i, 0)),
            pl.BlockSpec((bn, 32), lambda i: (i, 0)),
            pl.BlockSpec((C, C), lambda i: (0, 0)),
            pl.BlockSpec((1, C), lambda i: (0, 0)),
            pl.BlockSpec((1, C), lambda i: (0, 0)),
            pl.BlockSpec((1, C), lambda i: (0, 0)),
            pl.BlockSpec((C, C), lambda i: (0, 0)),
            pl.BlockSpec((C, C), lambda i: (0, 0)),
            pl.BlockSpec((1, C), lambda i: (0, 0)),
            pl.BlockSpec((1, C), lambda i: (0, 0)),
            pl.BlockSpec((1, C), lambda i: (0, 0)),
        ],
        out_specs=pl.BlockSpec((bn, C), lambda i: (i, 0)),
        out_shape=jax.ShapeDtypeStruct((N, C), f32),
    )(x, s_acc, ax_acc, att_W.T,
      att_b.reshape(1, C), norm_g.reshape(1, C), norm_b.reshape(1, C),
      fuse_t[:C], fuse_t[C:], fuse_b.reshape(1, C), fn_g.reshape(1, C),
      fn_b.reshape(1, C))

    return out.reshape(B, T, HW, C)


# speedup vs baseline: 2.1400x; 1.0005x over previous
import functools

import jax
import jax.numpy as jnp
from jax import lax
from jax.experimental import pallas as pl
from jax.experimental.pallas import tpu as pltpu
from jax.experimental.pallas import tpu_sc as plsc

NC = 2
NS = 16
NW = NC * NS
CH = 128
PADROWS = 512
DUPROWS = 128


def _gn(h, g, b, eps=1e-5):
    mu = jnp.mean(h, axis=-1, keepdims=True)
    d = h - mu
    var = jnp.mean(d * d, axis=-1, keepdims=True)
    return d * lax.rsqrt(var + eps) * g + b


def _lrelu(h):
    return jnp.where(h >= 0, h, 0.01 * h)


def _tc_edge_body(v_ref, pdx_ref, pdy_ref, w_ref,
                  d1t_ref, d1b_ref, d2t_ref, d2b_ref, dng_ref, dnb_ref,
                  u_ref, ax_ref):
    be = v_ref.shape[0]
    pdx = pdx_ref[...]
    pdy = pdy_ref[...]
    a = pdx * d1t_ref[0:1, :] + pdy * d1t_ref[1:2, :] + d1b_ref[...]
    h1 = _lrelu(a)
    h2 = jnp.dot(h1, d2t_ref[...], preferred_element_type=jnp.float32)
    h2 = h2 + d2b_ref[...]
    de = _gn(h2, dng_ref[...], dnb_ref[...])
    w = w_ref[...]
    u = w * (v_ref[...] + de)
    lane = lax.broadcasted_iota(jnp.int32, (be, 32), 1)
    aux = jnp.where(lane == 0, w, jnp.where(lane == 1, 1.0, 0.0))
    u_ref[...] = u
    ax_ref[...] = aux


def _tc_node_body(x_ref, s_ref, aux_ref,
                  attt_ref, attb_ref, ng_ref, nb_ref, ftx_ref, ftc_ref,
                  fb_ref, fng_ref, fnb_ref, out_ref):
    x = x_ref[...]
    aux = aux_ref[...]
    ws = aux[:, 0:1]
    cnt = aux[:, 1:2]
    s = s_ref[...]
    copy = jnp.dot(s, attt_ref[...], preferred_element_type=jnp.float32)
    copy = copy + ws * attb_ref[...]
    copy = _gn(copy, ng_ref[...], nb_ref[...])
    f = (jnp.dot(x, ftx_ref[...], preferred_element_type=jnp.float32)
         + jnp.dot(copy, ftc_ref[...], preferred_element_type=jnp.float32)
         + fb_ref[...])
    f = _lrelu(_gn(f, fng_ref[...], fnb_ref[...]))
    out_ref[...] = jnp.where(cnt > 0, f, x)


def _sc_gather_body(epad, x_hbm, ptab_hbm, pre_hbm, ip_hbm, is_hbm,
                    v_out, pdx_out, pdy_out,
                    ptab_v, preb, ipb, isb, vbuf, pdxb, pdyb, sem0):
    ew = epad // NW
    nchunk = ew // CH
    c = lax.axis_index("c")
    s = lax.axis_index("s")
    base = (s * NC + c) * ew

    pltpu.sync_copy(ptab_hbm, ptab_v)

    def body(k, carry):
        off = base + k * CH
        pltpu.sync_copy(pre_hbm.at[pl.ds(off, CH)], preb)
        pltpu.sync_copy(ip_hbm.at[pl.ds(off, CH)], ipb)
        pltpu.sync_copy(is_hbm.at[pl.ds(off, CH)], isb)
        cp0 = pltpu.async_copy(x_hbm.at[preb], vbuf, sem0)
        for g in range(CH // 16):
            ip_v = ipb[pl.ds(g * 16, 16)] * 2
            is_v = isb[pl.ds(g * 16, 16)] * 2
            ppx = plsc.load_gather(ptab_v, [ip_v])
            ppy = plsc.load_gather(ptab_v, [ip_v + 1])
            psx = plsc.load_gather(ptab_v, [is_v])
            psy = plsc.load_gather(ptab_v, [is_v + 1])
            pdxb[pl.ds(g * 16, 16)] = psx - ppx
            pdyb[pl.ds(g * 16, 16)] = psy - ppy
        pltpu.sync_copy(pdxb, pdx_out.at[pl.ds(off, CH)])
        pltpu.sync_copy(pdyb, pdy_out.at[pl.ds(off, CH)])
        cp0.wait()
        pltpu.sync_copy(vbuf, v_out.at[pl.ds(off, CH)])
        return carry

    lax.fori_loop(0, nchunk, body, 0)


def kernel(graph_feat, edge, weight, position, att_W, att_b, norm_g, norm_b,
           fuse_W, fuse_b, fn_g, fn_b, d1_W, d1_b, d2_W, d2_b, dn_g, dn_b):
    B, T, HW, C = graph_feat.shape
    N = B * T * HW
    E = edge.shape[0]
    THW = T * HW
    f32 = jnp.float32

    x = graph_feat.reshape(N, C)

    eb = edge[:, 0]
    et = edge[:, 1]
    ep = edge[:, 2]
    es = edge[:, 3]
    pre = eb * THW + et * HW + ep
    suc = eb * THW + (et + 1) * HW + es
    ip = et * HW + ep
    isc = (et + 1) * HW + es

    epad = ((E + NW * CH - 1) // (NW * CH)) * (NW * CH)
    npad = epad - E
    nacc = ((N + PADROWS + DUPROWS + 63) // 64) * 64
    ar = jnp.arange(npad, dtype=jnp.int32)
    pre_p = jnp.concatenate([pre, (ar * 97) % N])
    suc_p = jnp.concatenate([suc, N + (ar % PADROWS)])
    ip_p = jnp.concatenate([ip, ar % THW])
    is_p = jnp.concatenate([isc, ar % THW])
    w_p = jnp.concatenate([weight, jnp.zeros((npad,), f32)])

    ptab_flat = position.reshape(THW * 2)

    mesh = plsc.VectorSubcoreMesh(core_axis_name="c", subcore_axis_name="s",
                                  num_cores=NC, num_subcores=NS)

    sc_gather = pl.kernel(
        functools.partial(_sc_gather_body, epad),
        out_type=[
            jax.ShapeDtypeStruct((epad, C), f32),
            jax.ShapeDtypeStruct((epad,), f32),
            jax.ShapeDtypeStruct((epad,), f32),
        ],
        mesh=mesh,
        compiler_params=pltpu.CompilerParams(needs_layout_passes=False),
        scratch_types=[
            pltpu.VMEM((THW * 2,), f32),
            pltpu.VMEM((CH,), jnp.int32),
            pltpu.VMEM((CH,), jnp.int32),
            pltpu.VMEM((CH,), jnp.int32),
            pltpu.VMEM((CH, C), f32),
            pltpu.VMEM((CH,), f32),
            pltpu.VMEM((CH,), f32),
            pltpu.SemaphoreType.DMA,
        ],
    )
    v_rows, pdx, pdy = sc_gather(x, ptab_flat, pre_p, ip_p, is_p)

    be = 1024
    grid_e = epad // be
    u_full, aux32 = pl.pallas_call(
        _tc_edge_body,
        grid=(grid_e,),
        in_specs=[
            pl.BlockSpec((be, C), lambda i: (i, 0)),
            pl.BlockSpec((be, 1), lambda i: (i, 0)),
            pl.BlockSpec((be, 1), lambda i: (i, 0)),
            pl.BlockSpec((be, 1), lambda i: (i, 0)),
            pl.BlockSpec((2, C), lambda i: (0, 0)),
            pl.BlockSpec((1, C), lambda i: (0, 0)),
            pl.BlockSpec((C, C), lambda i: (0, 0)),
            pl.BlockSpec((1, C), lambda i: (0, 0)),
            pl.BlockSpec((1, C), lambda i: (0, 0)),
            pl.BlockSpec((1, C), lambda i: (0, 0)),
        ],
        out_specs=[pl.BlockSpec((be, C), lambda i: (i, 0)),
                   pl.BlockSpec((be, 32), lambda i: (i, 0))],
        out_shape=[jax.ShapeDtypeStruct((epad, C), f32),
                   jax.ShapeDtypeStruct((epad, 32), f32)],
    )(v_rows, pdx.reshape(epad, 1), pdy.reshape(epad, 1),
      w_p.reshape(epad, 1), d1_W.T, d1_b.reshape(1, C), d2_W.T,
      d2_b.reshape(1, C), dn_g.reshape(1, C), dn_b.reshape(1, C))

    s_acc = jnp.zeros((nacc, C), f32).at[suc_p].add(u_full)[:N]
    ax_acc = jnp.zeros((nacc, 32), f32).at[suc_p].add(aux32)[:N]

    bn = 1024
    grid_n = N // bn
    fuse_t = fuse_W.T
    out = pl.pallas_call(
        _tc_node_body,
        grid=(grid_n,),
        in_specs=[
            pl.BlockSpec((bn, C), lambda i: (i, 0)),
            pl.BlockSpec((bn, C), lambda i: (i, 0)),
            pl.BlockSpec((bn, 32), lambda i: (i, 0)),
            pl.BlockSpec((C, C), lambda i: (0, 0)),
            pl.BlockSpec((1, C), lambda i: (0, 0)),
            pl.BlockSpec((1, C), lambda i: (0, 0)),
            pl.BlockSpec((1, C), lambda i: (0, 0)),
            pl.BlockSpec((C, C), lambda i: (0, 0)),
            pl.BlockSpec((C, C), lambda i: (0, 0)),
            pl.BlockSpec((1, C), lambda i: (0, 0)),
            pl.BlockSpec((1, C), lambda i: (0, 0)),
            pl.BlockSpec((1, C), lambda i: (0, 0)),
        ],
        out_specs=pl.BlockSpec((bn, C), lambda i: (i, 0)),
        out_shape=jax.ShapeDtypeStruct((N, C), f32),
    )(x, s_acc, ax_acc, att_W.T,
      att_b.reshape(1, C), norm_g.reshape(1, C), norm_b.reshape(1, C),
      fuse_t[:C], fuse_t[C:], fuse_b.reshape(1, C), fn_g.reshape(1, C),
      fn_b.reshape(1, C))

    return out.reshape(B, T, HW, C)
